# Initial kernel scaffold; baseline (speedup 1.0000x reference)
import jax
import jax.numpy as jnp
from jax.experimental import pallas as pl


def kernel(x, edge_index, edge_attr, batch, B_total, lin1_W, lin1_b, W1a, b1a, W1b, b1b, lin2_W, lin2_b, W2a, b2a, W2b, b2b, Wr1, br1, Wr2, br2):
    raise NotImplementedError("write your pallas kernel here")


# speedup vs baseline: 1.2754x; 1.2754x over previous
import functools

import jax
import jax.numpy as jnp
from jax import lax
from jax.experimental import pallas as pl
from jax.experimental.pallas import tpu as pltpu
from jax.experimental.pallas import tpu_sc as plsc

NC = 2
NS = 16
NW = NC * NS
LANES = 16
RPT = 313
NPAD = NW * RPT
CAP = 24576
CHUNK = 128
SCAN = 2560
FPAD = 128
EPAD = 655360
EROWS = EPAD // 8
RELCH = 320
BIG = 2**30

_params = pltpu.CompilerParams(use_tc_tiling_on_sc=False,
                               needs_layout_passes=False)
_mesh = lambda: plsc.VectorSubcoreMesh(core_axis_name="c",
                                       subcore_axis_name="s")


def _sc_relayout():
    @functools.partial(
        pl.kernel,
        mesh=_mesh(),
        out_type=jax.ShapeDtypeStruct((EPAD, LANES), jnp.float32),
        compiler_params=_params,
        scratch_types=[
            pltpu.VMEM((RELCH, FPAD), jnp.float32),
            pltpu.VMEM((RELCH * 8, LANES), jnp.float32),
        ],
    )
    def rel(ea128_hbm, out_hbm, in_v, out_v):
        c = lax.axis_index("c")
        s = lax.axis_index("s")
        wid = s * NC + c
        slab = EROWS // NW

        def ch_body(t, carry):
            off = wid * slab + t * RELCH
            pltpu.sync_copy(ea128_hbm.at[pl.ds(off, RELCH)], in_v)

            def row_body(r, carry2):
                for j in range(8):
                    out_v[r * 8 + j, pl.ds(0, LANES)] = \
                        in_v[r, pl.ds(j * LANES, LANES)]
                return carry2

            lax.fori_loop(0, RELCH, row_body, 0)
            pltpu.sync_copy(out_v, out_hbm.at[pl.ds(off * 8, RELCH * 8)])
            return carry

        lax.fori_loop(0, slab // RELCH, ch_body, 0)

    return rel


def _make_sc_filter(n_edges: int):
    nscan = n_edges // SCAN
    assert n_edges % SCAN == 0

    @functools.partial(
        pl.kernel,
        mesh=_mesh(),
        out_type=(
            jax.ShapeDtypeStruct((NW, CAP), jnp.int32),
            jax.ShapeDtypeStruct((NW, CAP), jnp.int32),
            jax.ShapeDtypeStruct((NW, CAP), jnp.int32),
            jax.ShapeDtypeStruct((NW, LANES), jnp.int32),
        ),
        compiler_params=_params,
        scratch_types=[
            pltpu.VMEM((SCAN,), jnp.int32),
            pltpu.VMEM((SCAN,), jnp.int32),
            pltpu.VMEM((CAP,), jnp.int32),
            pltpu.VMEM((CAP,), jnp.int32),
            pltpu.VMEM((CAP,), jnp.int32),
            pltpu.VMEM((LANES,), jnp.int32),
        ],
    )
    def sc_filter(src_hbm, dst_hbm, eid_out, src_out, dst_out, cnt_out,
                  srcw_v, dstw_v, eid_v, srcl_v, dstl_v, cnt_v):
        c = lax.axis_index("c")
        s = lax.axis_index("s")
        wid = s * NC + c
        lo = wid * RPT
        hi = lo + RPT

        def fill_body(i, carry):
            eid_v[pl.ds(i * LANES, LANES)] = jnp.full((LANES,), n_edges,
                                                      jnp.int32)
            srcl_v[pl.ds(i * LANES, LANES)] = jnp.zeros((LANES,), jnp.int32)
            dstl_v[pl.ds(i * LANES, LANES)] = jnp.full((LANES,), BIG,
                                                       jnp.int32)
            return carry
        lax.fori_loop(0, CAP // LANES, fill_body, 0)

        def scan_body(t, pos):
            off = t * SCAN
            pltpu.sync_copy(src_hbm.at[pl.ds(off, SCAN)], srcw_v)
            pltpu.sync_copy(dst_hbm.at[pl.ds(off, SCAN)], dstw_v)

            def group_body(g, pos2):
                dvec = dstw_v[pl.ds(g * LANES, LANES)]
                svec = srcw_v[pl.ds(g * LANES, LANES)]
                mask = (dvec >= lo) & (dvec < hi)
                eids = lax.iota(jnp.int32, LANES) + (off + g * LANES)
                pfx = plsc.cumsum(mask.astype(jnp.int32))
                idx = pos2 + pfx - 1
                plsc.store_scatter(eid_v, [idx], eids, mask=mask)
                plsc.store_scatter(srcl_v, [idx], svec, mask=mask)
                plsc.store_scatter(dstl_v, [idx], dvec, mask=mask)
                return pos2 + pfx[LANES - 1]

            return lax.fori_loop(0, SCAN // LANES, group_body, pos)

        cnt = lax.fori_loop(0, nscan, scan_body, jnp.int32(0))
        cnt_v[...] = jnp.full((LANES,), 1, jnp.int32) * cnt
        pltpu.sync_copy(eid_v, eid_out.at[wid])
        pltpu.sync_copy(srcl_v, src_out.at[wid])
        pltpu.sync_copy(dstl_v, dst_out.at[wid])
        pltpu.sync_copy(cnt_v, cnt_out.at[wid])

    return sc_filter


def _make_sc_conv(feat: int):
    nj = feat // LANES

    @functools.partial(
        pl.kernel,
        mesh=_mesh(),
        out_type=(
            jax.ShapeDtypeStruct((NPAD, FPAD), jnp.float32),
            jax.ShapeDtypeStruct((NPAD, FPAD), jnp.float32),
        ),
        compiler_params=_params,
        scratch_types=[
            pltpu.VMEM((CHUNK,), jnp.int32),
            pltpu.VMEM((CHUNK,), jnp.int32),
            pltpu.VMEM((CHUNK,), jnp.int32),
            pltpu.VMEM((CHUNK, LANES), jnp.float32),
            pltpu.VMEM((CHUNK, FPAD), jnp.float32),
            pltpu.VMEM((5, FPAD), jnp.float32),
            pltpu.VMEM((LANES,), jnp.int32),
            pltpu.VMEM((RPT + 1, FPAD), jnp.float32),
            pltpu.VMEM((RPT + 1, FPAD), jnp.float32),
            pltpu.SemaphoreType.DMA,
            pltpu.SemaphoreType.DMA,
        ],
    )
    def sc_conv(x_hbm, eid_hbm, srcl_hbm, dstl_hbm, cnt_hbm, ea_hbm, w_hbm,
                z_hbm, s_out, c_out, eidw_v, srcw_v, dstw_v, ea_v, rows_v,
                w_v, cnt_v, acc_v, comp_v, sem, sem2):
        c = lax.axis_index("c")
        s = lax.axis_index("s")
        wid = s * NC + c
        lo = wid * RPT

        pltpu.sync_copy(cnt_hbm.at[wid], cnt_v)
        pltpu.sync_copy(w_hbm, w_v)
        pltpu.sync_copy(z_hbm.at[pl.ds(0, RPT + 1)], acc_v)
        pltpu.sync_copy(z_hbm.at[pl.ds(0, RPT + 1)], comp_v)
        cnt = cnt_v[pl.ds(0, LANES)][0]
        nchunk = (cnt + (CHUNK - 1)) // CHUNK

        wvec = [[w_v[k, pl.ds(j * LANES, LANES)] for j in range(nj)]
                for k in range(5)]

        def chunk_body(t, carry):
            base = t * CHUNK
            pltpu.sync_copy(eid_hbm.at[wid, pl.ds(base, CHUNK)], eidw_v)
            pltpu.sync_copy(srcl_hbm.at[wid, pl.ds(base, CHUNK)], srcw_v)
            pltpu.sync_copy(dstl_hbm.at[wid, pl.ds(base, CHUNK)], dstw_v)
            ea_cp = pltpu.async_copy(ea_hbm.at[eidw_v], ea_v, sem)
            x_cp = pltpu.async_copy(x_hbm.at[srcw_v], rows_v, sem2)
            ea_cp.wait()
            x_cp.wait()

            def group_body(g, carry2):
                dvec = dstw_v[pl.ds(g * LANES, LANES)] - lo
                dvec = jnp.clip(dvec, 0, RPT)
                for k in range(LANES):
                    i = g * LANES + k
                    dloc = dvec[k]
                    ev = ea_v[i, pl.ds(0, LANES)]
                    for j in range(nj):
                        r = rows_v[i, pl.ds(j * LANES, LANES)]
                        m = r + wvec[4][j]
                        m = m + ev[0] * wvec[0][j]
                        m = m + ev[1] * wvec[1][j]
                        m = m + ev[2] * wvec[2][j]
                        m = m + ev[3] * wvec[3][j]
                        m = jnp.maximum(m, 0.0)
                        sl = pl.ds(j * LANES, LANES)
                        sv = acc_v[dloc, sl]
                        cv = comp_v[dloc, sl]
                        y = m - cv
                        t2 = sv + y
                        comp_v[dloc, sl] = (t2 - sv) - y
                        acc_v[dloc, sl] = t2
                return carry2

            lax.fori_loop(0, CHUNK // LANES, group_body, 0)
            return carry

        lax.fori_loop(0, nchunk, chunk_body, 0)
        pltpu.sync_copy(acc_v.at[pl.ds(0, RPT)], s_out.at[pl.ds(lo, RPT)])
        pltpu.sync_copy(comp_v.at[pl.ds(0, RPT)], c_out.at[pl.ds(lo, RPT)])

    return sc_conv


def _bdot(a, b):
    return jnp.dot(a.astype(jnp.bfloat16), b.astype(jnp.bfloat16),
                   preferred_element_type=jnp.float32)


def _tc_mlp1(x, s1, c1, w1a, b1a, w1b, b1b):
    n, f = x.shape
    h = w1a.shape[1]

    def body(x_ref, s_ref, c_ref, wa_ref, ba_ref, wb_ref, bb_ref, o_ref):
        hh = x_ref[...] + (s_ref[...] + c_ref[...])
        hh = jnp.maximum(_bdot(hh, wa_ref[...]) + ba_ref[...], 0.0)
        hh = jnp.maximum(_bdot(hh, wb_ref[...]) + bb_ref[...], 0.0)
        o_ref[...] = jnp.concatenate(
            [hh, jnp.zeros((n, FPAD - h), jnp.float32)], axis=1)

    return pl.pallas_call(
        body,
        out_shape=jax.ShapeDtypeStruct((n, FPAD), jnp.float32),
    )(x, s1, c1, w1a, b1a.reshape(1, -1), w1b, b1b.reshape(1, -1))


def _tc_mlp2_softmax(h1p, s2, c2, w2a, b2a, w2b, b2b, wr1, br1, wr2, br2,
                     batch, b_total, n_graphs):
    n = h1p.shape[0]
    h = w2a.shape[0]

    def body(h_ref, s_ref, cc_ref, wa_ref, ba_ref, wb_ref, bb_ref,
             wr1_ref, br1_ref, wr2_ref, br2_ref, bat_ref, bt_ref, o_ref):
        hh = (h_ref[...] + (s_ref[...] + cc_ref[...]))[:, :h]
        hh = jnp.maximum(_bdot(hh, wa_ref[...]) + ba_ref[...], 0.0)
        hh = jnp.maximum(_bdot(hh, wb_ref[...]) + bb_ref[...], 0.0)
        r = jnp.maximum(_bdot(hh, wr1_ref[...]) + br1_ref[...], 0.0)
        rb = r.astype(jnp.bfloat16).astype(jnp.float32)
        w2b_ = wr2_ref[...].astype(jnp.bfloat16).astype(jnp.float32)
        raw = jnp.sum(rb * w2b_, axis=1, keepdims=True) + br2_ref[...]
        gid = lax.broadcasted_iota(jnp.int32, (n, n_graphs), 1)
        mask = gid == bat_ref[...]
        neg = jnp.float32(-3.0e38)
        m = jnp.max(jnp.where(mask, raw, neg), axis=0, keepdims=True)
        m_row = jnp.sum(jnp.where(mask, m, 0.0), axis=1, keepdims=True)
        ex = jnp.exp(raw - m_row)
        denom = jnp.sum(jnp.where(mask, ex, 0.0), axis=0, keepdims=True)
        den_row = jnp.sum(jnp.where(mask, denom, 0.0), axis=1, keepdims=True)
        b_row = jnp.sum(jnp.where(mask, bt_ref[...], 0.0), axis=1,
                        keepdims=True)
        o_ref[...] = ex / den_row * b_row

    return pl.pallas_call(
        body,
        out_shape=jax.ShapeDtypeStruct((n, 1), jnp.float32),
    )(h1p, s2, c2, w2a, b2a.reshape(1, -1), w2b, b2b.reshape(1, -1),
      wr1, br1.reshape(1, -1), wr2.reshape(1, -1), br2.reshape(1, 1),
      batch.reshape(-1, 1), b_total.reshape(1, -1))


def kernel(x, edge_index, edge_attr, batch, B_total,
           lin1_W, lin1_b, W1a, b1a, W1b, b1b,
           lin2_W, lin2_b, W2a, b2a, W2b, b2b,
           Wr1, br1, Wr2, br2):
    n, f_in = x.shape
    e = edge_index.shape[1]
    h = W1a.shape[1]
    g = B_total.shape[0]
    src = edge_index[0]
    dst = edge_index[1]

    def _r(a):
        return a.astype(jnp.bfloat16).astype(jnp.float32)

    def _aug(w, b):
        wp = jnp.pad(_r(w), ((0, 1), (0, FPAD - w.shape[1])))
        return wp.at[4, :b.shape[0]].set(b)

    w1_aug = _aug(lin1_W, lin1_b)
    w2_aug = _aug(lin2_W, lin2_b)
    ea128 = jnp.pad(_r(edge_attr),
                    ((0, EPAD - e), (0, LANES - edge_attr.shape[1])))
    ea128 = ea128.reshape(EROWS, FPAD)
    z = jnp.zeros((RPT + 1, FPAD), jnp.float32)

    ea16 = _sc_relayout()(ea128)
    eids, srcl, dstl, cnts = _make_sc_filter(e)(src, dst)

    s1, c1 = _make_sc_conv(f_in)(x, eids, srcl, dstl, cnts, ea16, w1_aug, z)
    h1p = _tc_mlp1(x, s1[:n], c1[:n], W1a, b1a, W1b, b1b)

    s2, c2 = _make_sc_conv(h)(h1p, eids, srcl, dstl, cnts, ea16, w2_aug, z)
    out = _tc_mlp2_softmax(h1p, s2[:n], c2[:n], W2a, b2a, W2b, b2b,
                           Wr1, br1, Wr2, br2, batch, B_total, g)
    return out[:, 0]


# speedup vs baseline: 1.3074x; 1.0251x over previous
import functools

import jax
import jax.numpy as jnp
from jax import lax
from jax.experimental import pallas as pl
from jax.experimental.pallas import tpu as pltpu
from jax.experimental.pallas import tpu_sc as plsc

NC = 2
NS = 16
NW = NC * NS
LANES = 16
RPT = 313
NPAD = NW * RPT
CAP = 24576
CHUNK = 128
SCAN = 2560
FPAD = 128
EPAD = 655360
EROWS = EPAD // 8
RELCH = 320
BIG = 2**30

_params = pltpu.CompilerParams(use_tc_tiling_on_sc=False,
                               needs_layout_passes=False)
_mesh = lambda: plsc.VectorSubcoreMesh(core_axis_name="c",
                                       subcore_axis_name="s")


def _sc_relayout():
    @functools.partial(
        pl.kernel,
        mesh=_mesh(),
        out_type=jax.ShapeDtypeStruct((EPAD, LANES), jnp.float32),
        compiler_params=_params,
        scratch_types=[
            pltpu.VMEM((RELCH, FPAD), jnp.float32),
            pltpu.VMEM((RELCH * 8, LANES), jnp.float32),
        ],
    )
    def rel(ea128_hbm, out_hbm, in_v, out_v):
        c = lax.axis_index("c")
        s = lax.axis_index("s")
        wid = s * NC + c
        slab = EROWS // NW

        def ch_body(t, carry):
            off = wid * slab + t * RELCH
            pltpu.sync_copy(ea128_hbm.at[pl.ds(off, RELCH)], in_v)

            def row_body(r, carry2):
                for j in range(8):
                    out_v[r * 8 + j, pl.ds(0, LANES)] = \
                        in_v[r, pl.ds(j * LANES, LANES)]
                return carry2

            lax.fori_loop(0, RELCH, row_body, 0)
            pltpu.sync_copy(out_v, out_hbm.at[pl.ds(off * 8, RELCH * 8)])
            return carry

        lax.fori_loop(0, slab // RELCH, ch_body, 0)

    return rel


def _make_sc_filter(n_edges: int):
    nscan = n_edges // SCAN
    assert n_edges % SCAN == 0

    @functools.partial(
        pl.kernel,
        mesh=_mesh(),
        out_type=(
            jax.ShapeDtypeStruct((NW, CAP), jnp.int32),
            jax.ShapeDtypeStruct((NW, CAP), jnp.int32),
            jax.ShapeDtypeStruct((NW, CAP), jnp.int32),
            jax.ShapeDtypeStruct((NW, LANES), jnp.int32),
        ),
        compiler_params=_params,
        scratch_types=[
            pltpu.VMEM((SCAN,), jnp.int32),
            pltpu.VMEM((SCAN,), jnp.int32),
            pltpu.VMEM((CAP,), jnp.int32),
            pltpu.VMEM((CAP,), jnp.int32),
            pltpu.VMEM((CAP,), jnp.int32),
            pltpu.VMEM((LANES,), jnp.int32),
        ],
    )
    def sc_filter(src_hbm, dst_hbm, eid_out, src_out, dst_out, cnt_out,
                  srcw_v, dstw_v, eid_v, srcl_v, dstl_v, cnt_v):
        c = lax.axis_index("c")
        s = lax.axis_index("s")
        wid = s * NC + c
        lo = wid * RPT
        hi = lo + RPT

        def fill_body(i, carry):
            eid_v[pl.ds(i * LANES, LANES)] = jnp.full((LANES,), n_edges,
                                                      jnp.int32)
            srcl_v[pl.ds(i * LANES, LANES)] = jnp.zeros((LANES,), jnp.int32)
            dstl_v[pl.ds(i * LANES, LANES)] = jnp.full((LANES,), BIG,
                                                       jnp.int32)
            return carry
        lax.fori_loop(0, CAP // LANES, fill_body, 0)

        def scan_body(t, pos):
            off = t * SCAN
            pltpu.sync_copy(src_hbm.at[pl.ds(off, SCAN)], srcw_v)
            pltpu.sync_copy(dst_hbm.at[pl.ds(off, SCAN)], dstw_v)

            def group_body(g, pos2):
                dvec = dstw_v[pl.ds(g * LANES, LANES)]
                svec = srcw_v[pl.ds(g * LANES, LANES)]
                mask = (dvec >= lo) & (dvec < hi)
                eids = lax.iota(jnp.int32, LANES) + (off + g * LANES)
                pfx = plsc.cumsum(mask.astype(jnp.int32))
                idx = pos2 + pfx - 1
                plsc.store_scatter(eid_v, [idx], eids, mask=mask)
                plsc.store_scatter(srcl_v, [idx], svec, mask=mask)
                plsc.store_scatter(dstl_v, [idx], dvec, mask=mask)
                return pos2 + pfx[LANES - 1]

            return lax.fori_loop(0, SCAN // LANES, group_body, pos)

        cnt = lax.fori_loop(0, nscan, scan_body, jnp.int32(0))
        cnt_v[...] = jnp.full((LANES,), 1, jnp.int32) * cnt
        pltpu.sync_copy(eid_v, eid_out.at[wid])
        pltpu.sync_copy(srcl_v, src_out.at[wid])
        pltpu.sync_copy(dstl_v, dst_out.at[wid])
        pltpu.sync_copy(cnt_v, cnt_out.at[wid])

    return sc_filter


def _make_sc_conv(feat: int):
    nj = feat // LANES

    @functools.partial(
        pl.kernel,
        mesh=_mesh(),
        out_type=(
            jax.ShapeDtypeStruct((NPAD, FPAD), jnp.float32),
            jax.ShapeDtypeStruct((NPAD, FPAD), jnp.float32),
        ),
        compiler_params=_params,
        scratch_types=[
            pltpu.VMEM((CHUNK,), jnp.int32),
            pltpu.VMEM((CHUNK,), jnp.int32),
            pltpu.VMEM((CHUNK,), jnp.int32),
            pltpu.VMEM((CHUNK, LANES), jnp.float32),
            pltpu.VMEM((CHUNK, FPAD), jnp.float32),
            pltpu.VMEM((5, FPAD), jnp.float32),
            pltpu.VMEM((LANES,), jnp.int32),
            pltpu.VMEM((RPT + 1, FPAD), jnp.float32),
            pltpu.VMEM((RPT + 1, FPAD), jnp.float32),
            pltpu.SemaphoreType.DMA,
            pltpu.SemaphoreType.DMA,
            pltpu.SemaphoreType.DMA,
        ],
    )
    def sc_conv(x_hbm, eid_hbm, srcl_hbm, dstl_hbm, cnt_hbm, ea_hbm, w_hbm,
                z_hbm, s_out, c_out, eidw_v, srcw_v, dstw_v, ea_v, rows_v,
                w_v, cnt_v, acc_v, comp_v, sem, sem2, sem3):
        c = lax.axis_index("c")
        s = lax.axis_index("s")
        wid = s * NC + c
        lo = wid * RPT

        pltpu.sync_copy(cnt_hbm.at[wid], cnt_v)
        pltpu.sync_copy(w_hbm, w_v)
        pltpu.sync_copy(z_hbm.at[pl.ds(0, RPT + 1)], acc_v)
        pltpu.sync_copy(z_hbm.at[pl.ds(0, RPT + 1)], comp_v)
        cnt = cnt_v[pl.ds(0, LANES)][0]
        nchunk = (cnt + (CHUNK - 1)) // CHUNK

        wvec = [[w_v[k, pl.ds(j * LANES, LANES)] for j in range(nj)]
                for k in range(5)]

        def chunk_body(t, carry):
            base = t * CHUNK
            e_cp = pltpu.async_copy(eid_hbm.at[wid, pl.ds(base, CHUNK)],
                                    eidw_v, sem3)
            s_cp = pltpu.async_copy(srcl_hbm.at[wid, pl.ds(base, CHUNK)],
                                    srcw_v, sem3)
            d_cp = pltpu.async_copy(dstl_hbm.at[wid, pl.ds(base, CHUNK)],
                                    dstw_v, sem3)
            e_cp.wait()
            s_cp.wait()
            d_cp.wait()
            ea_cp = pltpu.async_copy(ea_hbm.at[eidw_v], ea_v, sem)
            x_cp = pltpu.async_copy(x_hbm.at[srcw_v], rows_v, sem2)
            ea_cp.wait()
            x_cp.wait()

            def group_body(g, carry2):
                dvec = dstw_v[pl.ds(g * LANES, LANES)] - lo
                dvec = jnp.clip(dvec, 0, RPT)
                for k in range(LANES):
                    i = g * LANES + k
                    dloc = dvec[k]
                    ev = ea_v[i, pl.ds(0, LANES)]
                    for j in range(nj):
                        r = rows_v[i, pl.ds(j * LANES, LANES)]
                        m = r + wvec[4][j]
                        m = m + ev[0] * wvec[0][j]
                        m = m + ev[1] * wvec[1][j]
                        m = m + ev[2] * wvec[2][j]
                        m = m + ev[3] * wvec[3][j]
                        m = jnp.maximum(m, 0.0)
                        sl = pl.ds(j * LANES, LANES)
                        sv = acc_v[dloc, sl]
                        cv = comp_v[dloc, sl]
                        y = m - cv
                        t2 = sv + y
                        comp_v[dloc, sl] = (t2 - sv) - y
                        acc_v[dloc, sl] = t2
                return carry2

            lax.fori_loop(0, CHUNK // LANES, group_body, 0)
            return carry

        lax.fori_loop(0, nchunk, chunk_body, 0)
        pltpu.sync_copy(acc_v.at[pl.ds(0, RPT)], s_out.at[pl.ds(lo, RPT)])
        pltpu.sync_copy(comp_v.at[pl.ds(0, RPT)], c_out.at[pl.ds(lo, RPT)])

    return sc_conv


def _bdot(a, b):
    return jnp.dot(a.astype(jnp.bfloat16), b.astype(jnp.bfloat16),
                   preferred_element_type=jnp.float32)


def _tc_mlp1(x, s1, c1, w1a, b1a, w1b, b1b):
    n, f = x.shape
    h = w1a.shape[1]

    def body(x_ref, s_ref, c_ref, wa_ref, ba_ref, wb_ref, bb_ref, o_ref):
        hh = x_ref[...] + (s_ref[...] + c_ref[...])
        hh = jnp.maximum(_bdot(hh, wa_ref[...]) + ba_ref[...], 0.0)
        hh = jnp.maximum(_bdot(hh, wb_ref[...]) + bb_ref[...], 0.0)
        o_ref[...] = jnp.concatenate(
            [hh, jnp.zeros((n, FPAD - h), jnp.float32)], axis=1)

    return pl.pallas_call(
        body,
        out_shape=jax.ShapeDtypeStruct((n, FPAD), jnp.float32),
    )(x, s1, c1, w1a, b1a.reshape(1, -1), w1b, b1b.reshape(1, -1))


def _tc_mlp2_softmax(h1p, s2, c2, w2a, b2a, w2b, b2b, wr1, br1, wr2, br2,
                     batch, b_total, n_graphs):
    n = h1p.shape[0]
    h = w2a.shape[0]

    def body(h_ref, s_ref, cc_ref, wa_ref, ba_ref, wb_ref, bb_ref,
             wr1_ref, br1_ref, wr2_ref, br2_ref, bat_ref, bt_ref, o_ref):
        hh = (h_ref[...] + (s_ref[...] + cc_ref[...]))[:, :h]
        hh = jnp.maximum(_bdot(hh, wa_ref[...]) + ba_ref[...], 0.0)
        hh = jnp.maximum(_bdot(hh, wb_ref[...]) + bb_ref[...], 0.0)
        r = jnp.maximum(_bdot(hh, wr1_ref[...]) + br1_ref[...], 0.0)
        rb = r.astype(jnp.bfloat16).astype(jnp.float32)
        w2b_ = wr2_ref[...].astype(jnp.bfloat16).astype(jnp.float32)
        raw = jnp.sum(rb * w2b_, axis=1, keepdims=True) + br2_ref[...]
        gid = lax.broadcasted_iota(jnp.int32, (n, n_graphs), 1)
        mask = gid == bat_ref[...]
        neg = jnp.float32(-3.0e38)
        m = jnp.max(jnp.where(mask, raw, neg), axis=0, keepdims=True)
        m_row = jnp.sum(jnp.where(mask, m, 0.0), axis=1, keepdims=True)
        ex = jnp.exp(raw - m_row)
        denom = jnp.sum(jnp.where(mask, ex, 0.0), axis=0, keepdims=True)
        den_row = jnp.sum(jnp.where(mask, denom, 0.0), axis=1, keepdims=True)
        b_row = jnp.sum(jnp.where(mask, bt_ref[...], 0.0), axis=1,
                        keepdims=True)
        o_ref[...] = ex / den_row * b_row

    return pl.pallas_call(
        body,
        out_shape=jax.ShapeDtypeStruct((n, 1), jnp.float32),
    )(h1p, s2, c2, w2a, b2a.reshape(1, -1), w2b, b2b.reshape(1, -1),
      wr1, br1.reshape(1, -1), wr2.reshape(1, -1), br2.reshape(1, 1),
      batch.reshape(-1, 1), b_total.reshape(1, -1))


def kernel(x, edge_index, edge_attr, batch, B_total,
           lin1_W, lin1_b, W1a, b1a, W1b, b1b,
           lin2_W, lin2_b, W2a, b2a, W2b, b2b,
           Wr1, br1, Wr2, br2):
    n, f_in = x.shape
    e = edge_index.shape[1]
    h = W1a.shape[1]
    g = B_total.shape[0]
    src = edge_index[0]
    dst = edge_index[1]

    def _r(a):
        return a.astype(jnp.bfloat16).astype(jnp.float32)

    def _aug(w, b):
        wp = jnp.pad(_r(w), ((0, 1), (0, FPAD - w.shape[1])))
        return wp.at[4, :b.shape[0]].set(b)

    w1_aug = _aug(lin1_W, lin1_b)
    w2_aug = _aug(lin2_W, lin2_b)
    ea128 = jnp.pad(_r(edge_attr),
                    ((0, EPAD - e), (0, LANES - edge_attr.shape[1])))
    ea128 = ea128.reshape(EROWS, FPAD)
    z = jnp.zeros((RPT + 1, FPAD), jnp.float32)

    ea16 = _sc_relayout()(ea128)
    eids, srcl, dstl, cnts = _make_sc_filter(e)(src, dst)

    s1, c1 = _make_sc_conv(f_in)(x, eids, srcl, dstl, cnts, ea16, w1_aug, z)
    h1p = _tc_mlp1(x, s1[:n], c1[:n], W1a, b1a, W1b, b1b)

    s2, c2 = _make_sc_conv(h)(h1p, eids, srcl, dstl, cnts, ea16, w2_aug, z)
    out = _tc_mlp2_softmax(h1p, s2[:n], c2[:n], W2a, b2a, W2b, b2b,
                           Wr1, br1, Wr2, br2, batch, B_total, g)
    return out[:, 0]
